# Initial kernel scaffold; baseline (speedup 1.0000x reference)
import jax
import jax.numpy as jnp
from jax.experimental import pallas as pl


def kernel(hidden_states, W, b):
    raise NotImplementedError("write your pallas kernel here")


# speedup vs baseline: 5.4963x; 5.4963x over previous
import jax
import jax.numpy as jnp
from jax import lax
from jax.experimental import pallas as pl
from jax.experimental.pallas import tpu as pltpu

_K = 8


def _router_body(x_ref, w_ref, b_ref, scores_ref, idx_ref):
    x = x_ref[...]
    w = w_ref[...]
    logits = jnp.dot(x, w, preferred_element_type=jnp.float32) + b_ref[...]

    m_rows, n_exp = logits.shape
    iota = lax.broadcasted_iota(jnp.int32, (m_rows, n_exp), 1)
    neg_inf = jnp.float32(-jnp.inf)

    cur = logits
    vals = []
    idxs = []
    for _ in range(_K):
        m = jnp.max(cur, axis=-1, keepdims=True)
        at_max = cur == m
        idx = jnp.min(
            jnp.where(at_max, iota, jnp.int32(n_exp)), axis=-1, keepdims=True
        )
        cur = jnp.where(iota == idx, neg_inf, cur)
        vals.append(m)
        idxs.append(idx)

    exps = [jnp.exp(v - vals[0]) for v in vals]
    denom = exps[0]
    for e in exps[1:]:
        denom = denom + e
    inv = 1.0 / denom

    scores = jnp.zeros((m_rows, n_exp), jnp.float32)
    for k in range(_K):
        scores = jnp.where(iota == idxs[k], exps[k] * inv, scores)

    scores_ref[...] = scores
    idx_ref[...] = jnp.concatenate(idxs, axis=1)


def kernel(hidden_states, W, b):
    Bx, Sx, Hx = hidden_states.shape
    n_exp = W.shape[1]
    m_total = Bx * Sx
    x = hidden_states.reshape(m_total, Hx)
    b2 = b.reshape(1, n_exp)

    bm = 512 if m_total % 512 == 0 else m_total
    grid = (m_total // bm,)

    scores, indices = pl.pallas_call(
        _router_body,
        grid=grid,
        in_specs=[
            pl.BlockSpec((bm, Hx), lambda i: (i, 0)),
            pl.BlockSpec((Hx, n_exp), lambda i: (0, 0)),
            pl.BlockSpec((1, n_exp), lambda i: (0, 0)),
        ],
        out_specs=[
            pl.BlockSpec((bm, n_exp), lambda i: (i, 0)),
            pl.BlockSpec((bm, _K), lambda i: (i, 0)),
        ],
        out_shape=[
            jax.ShapeDtypeStruct((m_total, n_exp), jnp.float32),
            jax.ShapeDtypeStruct((m_total, _K), jnp.int32),
        ],
        compiler_params=pltpu.CompilerParams(
            dimension_semantics=("arbitrary",),
        ),
    )(x, W, b2)
    return scores, indices


# speedup vs baseline: 7.8852x; 1.4346x over previous
import jax
import jax.numpy as jnp
from jax import lax
from jax.experimental import pallas as pl
from jax.experimental.pallas import tpu as pltpu

_K = 8


def _router_body(x_ref, w_ref, b_ref, scores_ref, idx_ref):
    x = x_ref[...]
    w = w_ref[...]
    logits = jnp.dot(x, w, preferred_element_type=jnp.float32) + b_ref[...]
    lt = logits.T

    n_exp, bm = lt.shape
    iota_e = lax.broadcasted_iota(jnp.int32, (n_exp, bm), 0).astype(jnp.float32)
    neg_inf = jnp.float32(-jnp.inf)

    cur = lt
    vals = []
    idxs = []
    for _ in range(_K):
        m = jnp.max(cur, axis=0, keepdims=True)
        at_max = cur == m
        idx = jnp.min(
            jnp.where(at_max, iota_e, jnp.float32(n_exp)), axis=0, keepdims=True
        )
        cur = jnp.where(iota_e == idx, neg_inf, cur)
        vals.append(m)
        idxs.append(idx)

    chosen = cur == neg_inf
    m0 = vals[0]
    denom = jnp.exp(vals[0] - m0)
    for v in vals[1:]:
        denom = denom + jnp.exp(v - m0)
    inv = 1.0 / denom
    scores_t = jnp.where(chosen, jnp.exp(lt - m0) * inv, jnp.float32(0.0))
    scores_ref[...] = scores_t.T

    idx_t = jnp.concatenate(idxs, axis=0)
    idx_ref[...] = idx_t.T.astype(jnp.int32)


def kernel(hidden_states, W, b):
    Bx, Sx, Hx = hidden_states.shape
    n_exp = W.shape[1]
    m_total = Bx * Sx
    x = hidden_states.reshape(m_total, Hx)
    b2 = b.reshape(1, n_exp)

    bm = 512 if m_total % 512 == 0 else m_total
    grid = (m_total // bm,)

    scores, indices = pl.pallas_call(
        _router_body,
        grid=grid,
        in_specs=[
            pl.BlockSpec((bm, Hx), lambda i: (i, 0)),
            pl.BlockSpec((Hx, n_exp), lambda i: (0, 0)),
            pl.BlockSpec((1, n_exp), lambda i: (0, 0)),
        ],
        out_specs=[
            pl.BlockSpec((bm, n_exp), lambda i: (i, 0)),
            pl.BlockSpec((bm, _K), lambda i: (i, 0)),
        ],
        out_shape=[
            jax.ShapeDtypeStruct((m_total, n_exp), jnp.float32),
            jax.ShapeDtypeStruct((m_total, _K), jnp.int32),
        ],
        compiler_params=pltpu.CompilerParams(
            dimension_semantics=("arbitrary",),
        ),
    )(x, W, b2)
    return scores, indices


# speedup vs baseline: 9.1016x; 1.1543x over previous
import jax
import jax.numpy as jnp
from jax import lax
from jax.experimental import pallas as pl
from jax.experimental.pallas import tpu as pltpu

_K = 8


def _router_body(x_ref, w_ref, b_ref, scores_ref, idx_ref):
    x = x_ref[...]
    w = w_ref[...]
    logits = jnp.dot(x, w, preferred_element_type=jnp.float32) + b_ref[...]
    lt = logits.T

    n_exp, bm = lt.shape
    iota_e = lax.broadcasted_iota(jnp.int32, (n_exp, bm), 0).astype(jnp.float32)
    neg_inf = jnp.float32(-jnp.inf)

    cur = lt
    vals = []
    idxs = []
    for _ in range(_K):
        m = jnp.max(cur, axis=0, keepdims=True)
        at_max = cur == m
        idx = jnp.min(
            jnp.where(at_max, iota_e, jnp.float32(n_exp)), axis=0, keepdims=True
        )
        cur = jnp.where(iota_e == idx, neg_inf, cur)
        vals.append(m)
        idxs.append(idx)

    chosen = cur == neg_inf
    m0 = vals[0]
    denom = jnp.exp(vals[0] - m0)
    for v in vals[1:]:
        denom = denom + jnp.exp(v - m0)
    inv = 1.0 / denom
    scores_t = jnp.where(chosen, jnp.exp(lt - m0) * inv, jnp.float32(0.0))
    scores_ref[...] = scores_t.T

    idx_t = jnp.concatenate(idxs, axis=0)
    idx_ref[...] = idx_t.T.astype(jnp.int32)


def kernel(hidden_states, W, b):
    Bx, Sx, Hx = hidden_states.shape
    n_exp = W.shape[1]
    m_total = Bx * Sx
    x = hidden_states.reshape(m_total, Hx)
    b2 = b.reshape(1, n_exp)

    bm = 1024 if m_total % 1024 == 0 else m_total
    grid = (m_total // bm,)

    scores, indices = pl.pallas_call(
        _router_body,
        grid=grid,
        in_specs=[
            pl.BlockSpec((bm, Hx), lambda i: (i, 0)),
            pl.BlockSpec((Hx, n_exp), lambda i: (0, 0)),
            pl.BlockSpec((1, n_exp), lambda i: (0, 0)),
        ],
        out_specs=[
            pl.BlockSpec((bm, n_exp), lambda i: (i, 0)),
            pl.BlockSpec((bm, _K), lambda i: (i, 0)),
        ],
        out_shape=[
            jax.ShapeDtypeStruct((m_total, n_exp), jnp.float32),
            jax.ShapeDtypeStruct((m_total, _K), jnp.int32),
        ],
        compiler_params=pltpu.CompilerParams(
            dimension_semantics=("arbitrary",),
        ),
    )(x, W, b2)
    return scores, indices


# speedup vs baseline: 9.4566x; 1.0390x over previous
import jax
import jax.numpy as jnp
from jax import lax
from jax.experimental import pallas as pl
from jax.experimental.pallas import tpu as pltpu

_K = 8


def _router_body(x_ref, w_ref, b_ref, scores_ref, idx_ref):
    x = x_ref[...]
    w = w_ref[...]
    logits = jnp.dot(x, w, preferred_element_type=jnp.float32) + b_ref[...]
    lt = logits.T

    n_exp, bm = lt.shape
    iota_e = lax.broadcasted_iota(jnp.int32, (n_exp, bm), 0).astype(jnp.float32)
    neg_inf = jnp.float32(-jnp.inf)

    cur = lt
    vals = []
    idxs = []
    for _ in range(_K):
        m = jnp.max(cur, axis=0, keepdims=True)
        at_max = cur == m
        idx = jnp.min(
            jnp.where(at_max, iota_e, jnp.float32(n_exp)), axis=0, keepdims=True
        )
        cur = jnp.where(iota_e == idx, neg_inf, cur)
        vals.append(m)
        idxs.append(idx)

    chosen = cur == neg_inf
    m0 = vals[0]
    denom = jnp.exp(vals[0] - m0)
    for v in vals[1:]:
        denom = denom + jnp.exp(v - m0)
    inv = 1.0 / denom
    scores_t = jnp.where(chosen, jnp.exp(lt - m0) * inv, jnp.float32(0.0))
    scores_ref[...] = scores_t.T

    idx_t = jnp.concatenate(idxs, axis=0)
    idx_ref[...] = idx_t.T.astype(jnp.int32)


def kernel(hidden_states, W, b):
    Bx, Sx, Hx = hidden_states.shape
    n_exp = W.shape[1]
    m_total = Bx * Sx
    x = hidden_states.reshape(m_total, Hx)
    b2 = b.reshape(1, n_exp)

    bm = 2048 if m_total % 2048 == 0 else m_total
    grid = (m_total // bm,)

    scores, indices = pl.pallas_call(
        _router_body,
        grid=grid,
        in_specs=[
            pl.BlockSpec((bm, Hx), lambda i: (i, 0)),
            pl.BlockSpec((Hx, n_exp), lambda i: (0, 0)),
            pl.BlockSpec((1, n_exp), lambda i: (0, 0)),
        ],
        out_specs=[
            pl.BlockSpec((bm, n_exp), lambda i: (i, 0)),
            pl.BlockSpec((bm, _K), lambda i: (i, 0)),
        ],
        out_shape=[
            jax.ShapeDtypeStruct((m_total, n_exp), jnp.float32),
            jax.ShapeDtypeStruct((m_total, _K), jnp.int32),
        ],
        compiler_params=pltpu.CompilerParams(
            dimension_semantics=("arbitrary",),
        ),
    )(x, W, b2)
    return scores, indices
